# Initial kernel scaffold; baseline (speedup 1.0000x reference)
import jax
import jax.numpy as jnp
from jax.experimental import pallas as pl


def kernel(x, edge_index, W1, b1, W2, b2, W3, b3, Wf1, bf1, Wf2, bf2):
    raise NotImplementedError("write your pallas kernel here")


# speedup vs baseline: 9.7109x; 9.7109x over previous
import functools

import jax
import jax.numpy as jnp
from jax import lax
from jax.experimental import pallas as pl
from jax.experimental.pallas import tpu as pltpu
from jax.experimental.pallas import tpu_sc as plsc

NN = 10000
EE = 320000
DD = 128
N_PAD = 10240
ROWS_PER_TILE = N_PAD // 16
K = 128
NC, NS = 2, 16
NW = NC * NS
Q = -(-(EE // K) // NW)
E_PAD = NW * Q * K


def _sc_hist_body(dst_hbm, out_hbm, idx_v, ones_v, z_v, hist_sh):
    c = lax.axis_index("c")
    s = lax.axis_index("s")
    w = c * NS + s

    for j in range(ROWS_PER_TILE // 16):
        z_v[pl.ds(j * 16, 16)] = jnp.zeros((16,), jnp.float32)
    for j in range(K // 16):
        ones_v[pl.ds(j * 16, 16)] = jnp.ones((16,), jnp.float32)

    pltpu.sync_copy(z_v, hist_sh.at[pl.ds(s * ROWS_PER_TILE, ROWS_PER_TILE)])
    plsc.subcore_barrier()

    def step(q, _):
        base = (w * Q + q) * K
        pltpu.sync_copy(dst_hbm.at[pl.ds(base, K)], idx_v)
        pltpu.sync_copy(ones_v, hist_sh.at[idx_v], add=True)
        return 0

    lax.fori_loop(0, Q, step, 0)
    plsc.subcore_barrier()

    pltpu.sync_copy(hist_sh.at[pl.ds(s * ROWS_PER_TILE, ROWS_PER_TILE)],
                    out_hbm.at[c, pl.ds(s * ROWS_PER_TILE, ROWS_PER_TILE)])


_sc_hist = functools.partial(
    pl.kernel,
    out_type=jax.ShapeDtypeStruct((NC, N_PAD), jnp.float32),
    mesh=plsc.VectorSubcoreMesh(core_axis_name="c", subcore_axis_name="s"),
    scratch_types=[
        pltpu.VMEM((K,), jnp.int32),
        pltpu.VMEM((K,), jnp.float32),
        pltpu.VMEM((ROWS_PER_TILE,), jnp.float32),
        pltpu.VMEM_SHARED((N_PAD,), jnp.float32),
    ],
)(_sc_hist_body)


ZROWS = 64


def _sc_agg_body(hp_hbm, src_hbm, dst_hbm, out_hbm,
                 sidx_v, didx_v, rows_v, z_v, agg_sh, sem):
    c = lax.axis_index("c")
    s = lax.axis_index("s")
    w = c * NS + s

    def zrow(i, _):
        for j in range(DD // 16):
            z_v[i, pl.ds(j * 16, 16)] = jnp.zeros((16,), jnp.float32)
        return 0

    lax.fori_loop(0, ZROWS, zrow, 0)
    for t in range(ROWS_PER_TILE // ZROWS):
        pltpu.sync_copy(
            z_v, agg_sh.at[pl.ds(s * ROWS_PER_TILE + t * ZROWS, ZROWS), :])
    plsc.subcore_barrier()

    def step(q, _):
        base = (w * Q + q) * K
        pltpu.sync_copy(src_hbm.at[pl.ds(base, K)], sidx_v)
        pltpu.sync_copy(dst_hbm.at[pl.ds(base, K)], didx_v)
        pltpu.async_copy(hp_hbm.at[sidx_v], rows_v, sem).wait()
        pltpu.sync_copy(rows_v, agg_sh.at[didx_v], add=True)
        return 0

    lax.fori_loop(0, Q, step, 0)
    plsc.subcore_barrier()

    pltpu.sync_copy(
        agg_sh.at[pl.ds(s * ROWS_PER_TILE, ROWS_PER_TILE), :],
        out_hbm.at[c, pl.ds(s * ROWS_PER_TILE, ROWS_PER_TILE), :])


_sc_agg = functools.partial(
    pl.kernel,
    out_type=jax.ShapeDtypeStruct((NC, N_PAD, DD), jnp.float32),
    mesh=plsc.VectorSubcoreMesh(core_axis_name="c", subcore_axis_name="s"),
    scratch_types=[
        pltpu.VMEM((K,), jnp.int32),
        pltpu.VMEM((K,), jnp.int32),
        pltpu.VMEM((K, DD), jnp.float32),
        pltpu.VMEM((ZROWS, DD), jnp.float32),
        pltpu.VMEM_SHARED((N_PAD, DD), jnp.float32),
        pltpu.SemaphoreType.DMA,
    ],
)(_sc_agg_body)


RB = 1000


def _tc_first_body(x_ref, w_ref, ph_ref, hp_ref, dinv_ref):
    deg = 1.0 + ph_ref[0] + ph_ref[1]
    dv = lax.rsqrt(deg)
    h = jnp.dot(x_ref[...], w_ref[...], preferred_element_type=jnp.float32)
    hp_ref[...] = h * dv
    dinv_ref[...] = dv


def _tc_first(x, w1, ph):
    return pl.pallas_call(
        _tc_first_body,
        grid=(NN // RB,),
        in_specs=[
            pl.BlockSpec((RB, DD), lambda i: (i, 0)),
            pl.BlockSpec((DD, DD), lambda i: (0, 0)),
            pl.BlockSpec((NC, RB, 1), lambda i: (0, i, 0)),
        ],
        out_specs=[
            pl.BlockSpec((RB, DD), lambda i: (i, 0)),
            pl.BlockSpec((RB, 1), lambda i: (i, 0)),
        ],
        out_shape=[
            jax.ShapeDtypeStruct((NN, DD), jnp.float32),
            jax.ShapeDtypeStruct((NN, 1), jnp.float32),
        ],
    )(x, w1, ph)


def _tc_layer_body(p_ref, hp_ref, dinv_ref, b_ref, w_ref, out_ref):
    dv = dinv_ref[...]
    acc = p_ref[0] + p_ref[1] + hp_ref[...]
    xx = jnp.maximum(acc * dv + b_ref[...], 0.0)
    out_ref[...] = dv * jnp.dot(xx, w_ref[...],
                                preferred_element_type=jnp.float32)


def _tc_layer(p, hp, dinv, b, w):
    return pl.pallas_call(
        _tc_layer_body,
        grid=(NN // RB,),
        in_specs=[
            pl.BlockSpec((NC, RB, DD), lambda i: (0, i, 0)),
            pl.BlockSpec((RB, DD), lambda i: (i, 0)),
            pl.BlockSpec((RB, 1), lambda i: (i, 0)),
            pl.BlockSpec((1, DD), lambda i: (0, 0)),
            pl.BlockSpec((DD, DD), lambda i: (0, 0)),
        ],
        out_specs=pl.BlockSpec((RB, DD), lambda i: (i, 0)),
        out_shape=jax.ShapeDtypeStruct((NN, DD), jnp.float32),
    )(p, hp, dinv, b, w)


def _tc_head_body(p_ref, hp_ref, dinv_ref, b_ref, wf1_ref, bf1_ref,
                  wf2_ref, bf2_ref, out_ref):
    dv = dinv_ref[...]
    acc = p_ref[0] + p_ref[1] + hp_ref[...]
    xx = jnp.maximum(acc * dv + b_ref[...], 0.0)
    hh = jnp.maximum(
        jnp.dot(xx, wf1_ref[...], preferred_element_type=jnp.float32)
        + bf1_ref[...], 0.0)
    out_ref[...] = jnp.dot(hh, wf2_ref[...],
                           preferred_element_type=jnp.float32) + bf2_ref[...]


def _tc_head(p, hp, dinv, b, wf1, bf1, wf2, bf2):
    return pl.pallas_call(
        _tc_head_body,
        grid=(NN // RB,),
        in_specs=[
            pl.BlockSpec((NC, RB, DD), lambda i: (0, i, 0)),
            pl.BlockSpec((RB, DD), lambda i: (i, 0)),
            pl.BlockSpec((RB, 1), lambda i: (i, 0)),
            pl.BlockSpec((1, DD), lambda i: (0, 0)),
            pl.BlockSpec((DD, DD), lambda i: (0, 0)),
            pl.BlockSpec((1, DD), lambda i: (0, 0)),
            pl.BlockSpec((DD, DD), lambda i: (0, 0)),
            pl.BlockSpec((1, DD), lambda i: (0, 0)),
        ],
        out_specs=pl.BlockSpec((RB, DD), lambda i: (i, 0)),
        out_shape=jax.ShapeDtypeStruct((NN, DD), jnp.float32),
    )(p, hp, dinv, b, wf1, bf1, wf2, bf2)


def kernel(x, edge_index, W1, b1, W2, b2, W3, b3, Wf1, bf1, Wf2, bf2):
    src = edge_index[0]
    dst = edge_index[1]
    pad = E_PAD - EE
    srcp = jnp.concatenate([src, jnp.zeros((pad,), jnp.int32)])
    dstp = jnp.concatenate([dst, jnp.full((pad,), NN, jnp.int32)])

    ph = _sc_hist(dstp).reshape(NC, N_PAD, 1)
    hp1, dinv = _tc_first(x, W1, ph)

    b1r = b1.reshape(1, DD)
    b2r = b2.reshape(1, DD)
    b3r = b3.reshape(1, DD)
    bf1r = bf1.reshape(1, DD)
    wf2p = jnp.pad(Wf2, ((0, 0), (0, DD - Wf2.shape[1])))
    bf2p = jnp.pad(bf2, (0, DD - bf2.shape[0])).reshape(1, DD)

    p1 = _sc_agg(hp1, srcp, dstp)
    hp2 = _tc_layer(p1, hp1, dinv, b1r, W2)
    p2 = _sc_agg(hp2, srcp, dstp)
    hp3 = _tc_layer(p2, hp2, dinv, b2r, W3)
    p3 = _sc_agg(hp3, srcp, dstp)
    out = _tc_head(p3, hp3, dinv, b3r, Wf1, bf1r, wf2p, bf2p)
    return out[:, :Wf2.shape[1]]
